# baseline (speedup 1.0000x reference)
import functools

import jax
import jax.numpy as jnp
from jax import lax
from jax.experimental import pallas as pl
from jax.experimental.pallas import tpu as pltpu
from jax.experimental.pallas import tpu_sc as plsc

B, C, NY, NX = 4, 64, 496, 432
N_PILLARS = 48000
PLANE = NY * NX
TOT = B * PLANE
LANES = 16

NC, NS = 2, 16
NW = NC * NS
POS_PER_W = TOT // NW

TW = 48128
SENT = N_PILLARS

PCHUNK = 9600
K1_CHUNKS = N_PILLARS // PCHUNK
K1_GB = 6
K1_STEPS = PCHUNK // (LANES * K1_GB)

CROWS = 8
CP = CROWS * NY
K2_CHUNKS = NX // CROWS
K2_NCH = B * K2_CHUNKS
K2_GROUPS = NY // LANES

_mesh = plsc.VectorSubcoreMesh(core_axis_name="c", subcore_axis_name="s")
_sc_params = pltpu.CompilerParams(needs_layout_passes=False)


def _wid():
    return lax.axis_index("s") * NC + lax.axis_index("c")


def _tr_body(vf_ref, out_ref):
    i = pl.program_id(0)

    @pl.when(i < N_PILLARS // 128)
    def _():
        out_ref[...] = vf_ref[...].T

    @pl.when(i >= N_PILLARS // 128)
    def _():
        out_ref[...] = jnp.zeros((C, 128), jnp.float32)


def _transpose_table(vf):
    return pl.pallas_call(
        _tr_body,
        grid=(TW // 128,),
        in_specs=[pl.BlockSpec((128, C), lambda i: (jnp.minimum(i, N_PILLARS // 128 - 1), 0))],
        out_specs=pl.BlockSpec((C, 128), lambda i: (0, i)),
        out_shape=jax.ShapeDtypeStruct((C, TW), jnp.float32),
    )(vf)


@functools.partial(
    pl.kernel,
    out_type=jax.ShapeDtypeStruct((TOT,), jnp.int32),
    mesh=_mesh,
    compiler_params=_sc_params,
    scratch_types=[
        pltpu.VMEM((POS_PER_W,), jnp.int32),
        pltpu.VMEM((2, PCHUNK), jnp.int32),
        pltpu.VMEM((2, PCHUNK), jnp.int32),
        pltpu.VMEM((2, PCHUNK), jnp.int32),
        pltpu.SemaphoreType.DMA,
        pltpu.SemaphoreType.DMA,
    ],
)
def _build_pid(b_hbm, y_hbm, x_hbm, pid_hbm, slab, bv, yv, xv, si0, si1):
    wid = _wid()
    lo = wid * POS_PER_W
    lane = lax.iota(jnp.int32, LANES)
    sent_vec = SENT + lane
    sin = (si0, si1)

    def start_in(ci, q):
        base = ci * PCHUNK
        pltpu.async_copy(b_hbm.at[pl.ds(base, PCHUNK)], bv.at[q], sin[q])
        pltpu.async_copy(y_hbm.at[pl.ds(base, PCHUNK)], yv.at[q], sin[q])
        pltpu.async_copy(x_hbm.at[pl.ds(base, PCHUNK)], xv.at[q], sin[q])

    def wait_in(q):
        pltpu.make_async_copy(b_hbm.at[pl.ds(0, PCHUNK)], bv.at[q], sin[q]).wait()
        pltpu.make_async_copy(y_hbm.at[pl.ds(0, PCHUNK)], yv.at[q], sin[q]).wait()
        pltpu.make_async_copy(x_hbm.at[pl.ds(0, PCHUNK)], xv.at[q], sin[q]).wait()

    start_in(0, 0)
    start_in(1, 1)

    def ibody(i, carry):
        slab[pl.ds(i * LANES, LANES)] = sent_vec
        return carry

    lax.fori_loop(0, POS_PER_W // LANES, ibody, 0, unroll=8)

    for ci in range(K1_CHUNKS):
        q = ci & 1
        base = ci * PCHUNK
        wait_in(q)

        def gbody(t, carry):
            offs = [(t * K1_GB + j) * LANES for j in range(K1_GB)]
            bbs = [bv[q, pl.ds(o, LANES)] for o in offs]
            yys = [yv[q, pl.ds(o, LANES)] for o in offs]
            xxs = [xv[q, pl.ds(o, LANES)] for o in offs]
            for o, bb, yy, xx in zip(offs, bbs, yys, xxs):
                flat = bb * PLANE + xx * NY + yy
                loc = flat - lo
                mask = (loc >= 0) & (loc < POS_PER_W)
                safe = jnp.where(mask, loc, 0)
                pidv = (base + o) + lane
                plsc.store_scatter(slab, [safe], pidv, mask=mask)
            return carry

        lax.fori_loop(0, K1_STEPS, gbody, 0)
        if ci + 2 < K1_CHUNKS:
            start_in(ci + 2, q)

    pltpu.sync_copy(slab, pid_hbm.at[pl.ds(lo, POS_PER_W)])


@functools.partial(
    pl.kernel,
    out_type=jax.ShapeDtypeStruct((B, C, NX, NY), jnp.float32),
    mesh=_mesh,
    compiler_params=_sc_params,
    scratch_types=[
        pltpu.VMEM((TW,), jnp.float32),
        pltpu.VMEM((TW,), jnp.float32),
        pltpu.VMEM((2, CP), jnp.int32),
        pltpu.VMEM((2, CROWS, NY), jnp.float32),
        pltpu.VMEM((2, CROWS, NY), jnp.float32),
        pltpu.SemaphoreType.DMA,
        pltpu.SemaphoreType.DMA,
        pltpu.SemaphoreType.DMA,
        pltpu.SemaphoreType.DMA,
    ],
)
def _expand(pid_hbm, tab_hbm, out_hbm, t0, t1, pidb, ob0, ob1, si0, si1, so0, so1):
    wid = _wid()
    c0 = wid * 2
    pltpu.sync_copy(tab_hbm.at[c0], t0)
    pltpu.sync_copy(tab_hbm.at[c0 + 1], t1)

    sin = (si0, si1)
    sout = (so0, so1)

    def bk(i):
        b = i // K2_CHUNKS
        return b, i - b * K2_CHUNKS

    def start_in(i, q):
        b, k = bk(i)
        pltpu.async_copy(pid_hbm.at[pl.ds(b * PLANE + k * CP, CP)], pidb.at[q], sin[q])

    def wait_in(q):
        pltpu.make_async_copy(pid_hbm.at[pl.ds(0, CP)], pidb.at[q], sin[q]).wait()

    def start_out(i, q):
        b, k = bk(i)
        pltpu.async_copy(ob0.at[q], out_hbm.at[b, c0, pl.ds(k * CROWS, CROWS)], sout[q])
        pltpu.async_copy(ob1.at[q], out_hbm.at[b, c0 + 1, pl.ds(k * CROWS, CROWS)], sout[q])

    def wait_out(q):
        pltpu.make_async_copy(ob0.at[q], out_hbm.at[0, 0, pl.ds(0, CROWS)], sout[q]).wait()
        pltpu.make_async_copy(ob1.at[q], out_hbm.at[0, 0, pl.ds(0, CROWS)], sout[q]).wait()

    def compute(q):
        def rbody(r, carry):
            rb = r * NY
            for q0, qn in ((0, 8), (8, 8), (16, 8), (24, 7)):
                offs = [(q0 + j) * LANES for j in range(qn)]
                idxs = [pidb[q, pl.ds(rb + o, LANES)] for o in offs]
                v0s = [plsc.load_gather(t0, [ix]) for ix in idxs]
                v1s = [plsc.load_gather(t1, [ix]) for ix in idxs]
                for o, v0, v1 in zip(offs, v0s, v1s):
                    ob0[q, r, pl.ds(o, LANES)] = v0
                    ob1[q, r, pl.ds(o, LANES)] = v1
            return carry

        lax.fori_loop(0, CROWS, rbody, 0)

    start_in(0, 0)
    start_in(1, 1)
    for q in (0, 1):
        wait_in(q)
        compute(q)
        start_out(q, q)
        start_in(q + 2, q)

    def pbody(j, carry):
        for q in (0, 1):
            i = 2 + 2 * j + q
            wait_in(q)
            wait_out(q)
            compute(q)
            start_out(i, q)
            start_in(jnp.minimum(i + 2, K2_NCH - 1), q)
        return carry

    lax.fori_loop(0, (K2_NCH - 2) // 2, pbody, 0)

    for q in (0, 1):
        wait_in(q)
        wait_out(q)


def kernel(voxel_features, batch_idx, y_idx, x_idx):
    vf = voxel_features.astype(jnp.float32)
    bi = batch_idx.astype(jnp.int32)
    yi = y_idx.astype(jnp.int32)
    xi = x_idx.astype(jnp.int32)

    table = _transpose_table(vf)
    pid = _build_pid(bi, yi, xi)
    return jnp.swapaxes(_expand(pid, table), 2, 3)


# speedup vs baseline: 1.5062x; 1.5062x over previous
import functools

import jax
import jax.numpy as jnp
from jax import lax
from jax.experimental import pallas as pl
from jax.experimental.pallas import tpu as pltpu
from jax.experimental.pallas import tpu_sc as plsc

B, C, NY, NX = 4, 64, 496, 432
N_PILLARS = 48000
PLANE = NY * NX
TOT = B * PLANE
LANES = 16

NC, NS = 2, 16
NW = NC * NS
POS_PER_W = TOT // NW

TW = 48128
SENT = N_PILLARS

PCHUNK = 9600
K1_CHUNKS = N_PILLARS // PCHUNK
K1_GB = 6
K1_STEPS = PCHUNK // (LANES * K1_GB)

CROWS = 8
CP = CROWS * NY
K2_CHUNKS = NX // CROWS
K2_NCH = B * K2_CHUNKS
K2_GROUPS = NY // LANES

_mesh = plsc.VectorSubcoreMesh(core_axis_name="c", subcore_axis_name="s")
_sc_params = pltpu.CompilerParams(needs_layout_passes=False)


def _wid():
    return lax.axis_index("s") * NC + lax.axis_index("c")


K0_BLK = 1024


def _tr_body(vf_ref, out_ref):
    i = pl.program_id(0)
    rows = i * K0_BLK + lax.broadcasted_iota(jnp.int32, (K0_BLK, 1), 0)
    x = jnp.where(rows < N_PILLARS, vf_ref[...], 0.0)
    out_ref[...] = x.T


def _transpose_table(vf):
    return pl.pallas_call(
        _tr_body,
        grid=(TW // K0_BLK,),
        in_specs=[pl.BlockSpec((K0_BLK, C), lambda i: (i, 0))],
        out_specs=pl.BlockSpec((C, K0_BLK), lambda i: (0, i)),
        out_shape=jax.ShapeDtypeStruct((C, TW), jnp.float32),
    )(vf)


@functools.partial(
    pl.kernel,
    out_type=jax.ShapeDtypeStruct((TOT,), jnp.int32),
    mesh=_mesh,
    compiler_params=_sc_params,
    scratch_types=[
        pltpu.VMEM((POS_PER_W,), jnp.int32),
        pltpu.VMEM((2, PCHUNK), jnp.int32),
        pltpu.VMEM((2, PCHUNK), jnp.int32),
        pltpu.VMEM((2, PCHUNK), jnp.int32),
        pltpu.SemaphoreType.DMA,
        pltpu.SemaphoreType.DMA,
    ],
)
def _build_pid(b_hbm, y_hbm, x_hbm, pid_hbm, slab, bv, yv, xv, si0, si1):
    wid = _wid()
    lo = wid * POS_PER_W
    lane = lax.iota(jnp.int32, LANES)
    sent_vec = SENT + lane
    sin = (si0, si1)

    def start_in(ci, q):
        base = ci * PCHUNK
        pltpu.async_copy(b_hbm.at[pl.ds(base, PCHUNK)], bv.at[q], sin[q])
        pltpu.async_copy(y_hbm.at[pl.ds(base, PCHUNK)], yv.at[q], sin[q])
        pltpu.async_copy(x_hbm.at[pl.ds(base, PCHUNK)], xv.at[q], sin[q])

    def wait_in(q):
        pltpu.make_async_copy(b_hbm.at[pl.ds(0, PCHUNK)], bv.at[q], sin[q]).wait()
        pltpu.make_async_copy(y_hbm.at[pl.ds(0, PCHUNK)], yv.at[q], sin[q]).wait()
        pltpu.make_async_copy(x_hbm.at[pl.ds(0, PCHUNK)], xv.at[q], sin[q]).wait()

    start_in(0, 0)
    start_in(1, 1)

    def ibody(i, carry):
        slab[pl.ds(i * LANES, LANES)] = sent_vec
        return carry

    lax.fori_loop(0, POS_PER_W // LANES, ibody, 0, unroll=8)

    for ci in range(K1_CHUNKS):
        q = ci & 1
        base = ci * PCHUNK
        wait_in(q)

        def gbody(t, carry):
            offs = [(t * K1_GB + j) * LANES for j in range(K1_GB)]
            bbs = [bv[q, pl.ds(o, LANES)] for o in offs]
            yys = [yv[q, pl.ds(o, LANES)] for o in offs]
            xxs = [xv[q, pl.ds(o, LANES)] for o in offs]
            for o, bb, yy, xx in zip(offs, bbs, yys, xxs):
                flat = bb * PLANE + xx * NY + yy
                loc = flat - lo
                mask = (loc >= 0) & (loc < POS_PER_W)
                safe = jnp.where(mask, loc, 0)
                pidv = (base + o) + lane
                plsc.store_scatter(slab, [safe], pidv, mask=mask)
            return carry

        lax.fori_loop(0, K1_STEPS, gbody, 0)
        if ci + 2 < K1_CHUNKS:
            start_in(ci + 2, q)

    pltpu.sync_copy(slab, pid_hbm.at[pl.ds(lo, POS_PER_W)])


@functools.partial(
    pl.kernel,
    out_type=jax.ShapeDtypeStruct((B, C, NX, NY), jnp.float32),
    mesh=_mesh,
    compiler_params=_sc_params,
    scratch_types=[
        pltpu.VMEM((TW,), jnp.float32),
        pltpu.VMEM((TW,), jnp.float32),
        pltpu.VMEM((2, CP), jnp.int32),
        pltpu.VMEM((2, CROWS, NY), jnp.float32),
        pltpu.VMEM((2, CROWS, NY), jnp.float32),
        pltpu.SemaphoreType.DMA,
        pltpu.SemaphoreType.DMA,
        pltpu.SemaphoreType.DMA,
        pltpu.SemaphoreType.DMA,
    ],
)
def _expand(pid_hbm, tab_hbm, out_hbm, t0, t1, pidb, ob0, ob1, si0, si1, so0, so1):
    wid = _wid()
    c0 = wid * 2
    pltpu.sync_copy(tab_hbm.at[c0], t0)
    pltpu.sync_copy(tab_hbm.at[c0 + 1], t1)

    sin = (si0, si1)
    sout = (so0, so1)

    def bk(i):
        b = i // K2_CHUNKS
        return b, i - b * K2_CHUNKS

    def start_in(i, q):
        b, k = bk(i)
        pltpu.async_copy(pid_hbm.at[pl.ds(b * PLANE + k * CP, CP)], pidb.at[q], sin[q])

    def wait_in(q):
        pltpu.make_async_copy(pid_hbm.at[pl.ds(0, CP)], pidb.at[q], sin[q]).wait()

    def start_out(i, q):
        b, k = bk(i)
        pltpu.async_copy(ob0.at[q], out_hbm.at[b, c0, pl.ds(k * CROWS, CROWS)], sout[q])
        pltpu.async_copy(ob1.at[q], out_hbm.at[b, c0 + 1, pl.ds(k * CROWS, CROWS)], sout[q])

    def wait_out(q):
        pltpu.make_async_copy(ob0.at[q], out_hbm.at[0, 0, pl.ds(0, CROWS)], sout[q]).wait()
        pltpu.make_async_copy(ob1.at[q], out_hbm.at[0, 0, pl.ds(0, CROWS)], sout[q]).wait()

    def compute(q):
        def rbody(r, carry):
            rb = r * NY
            for q0, qn in ((0, 8), (8, 8), (16, 8), (24, 7)):
                offs = [(q0 + j) * LANES for j in range(qn)]
                idxs = [pidb[q, pl.ds(rb + o, LANES)] for o in offs]
                v0s = [plsc.load_gather(t0, [ix]) for ix in idxs]
                v1s = [plsc.load_gather(t1, [ix]) for ix in idxs]
                for o, v0, v1 in zip(offs, v0s, v1s):
                    ob0[q, r, pl.ds(o, LANES)] = v0
                    ob1[q, r, pl.ds(o, LANES)] = v1
            return carry

        lax.fori_loop(0, CROWS, rbody, 0)

    start_in(0, 0)
    start_in(1, 1)
    for q in (0, 1):
        wait_in(q)
        compute(q)
        start_out(q, q)
        start_in(q + 2, q)

    def pbody(j, carry):
        for q in (0, 1):
            i = 2 + 2 * j + q
            wait_in(q)
            wait_out(q)
            compute(q)
            start_out(i, q)
            start_in(jnp.minimum(i + 2, K2_NCH - 1), q)
        return carry

    lax.fori_loop(0, (K2_NCH - 2) // 2, pbody, 0)

    for q in (0, 1):
        wait_in(q)
        wait_out(q)


def kernel(voxel_features, batch_idx, y_idx, x_idx):
    vf = voxel_features.astype(jnp.float32)
    bi = batch_idx.astype(jnp.int32)
    yi = y_idx.astype(jnp.int32)
    xi = x_idx.astype(jnp.int32)

    table = _transpose_table(vf)
    pid = _build_pid(bi, yi, xi)
    return jnp.swapaxes(_expand(pid, table), 2, 3)


# speedup vs baseline: 1.5118x; 1.0038x over previous
import functools

import jax
import jax.numpy as jnp
from jax import lax
from jax.experimental import pallas as pl
from jax.experimental.pallas import tpu as pltpu
from jax.experimental.pallas import tpu_sc as plsc

B, C, NY, NX = 4, 64, 496, 432
N_PILLARS = 48000
PLANE = NY * NX
TOT = B * PLANE
LANES = 16

NC, NS = 2, 16
NW = NC * NS
POS_PER_W = TOT // NW

TW = 48128
SENT = N_PILLARS

PCHUNK = 9600
K1_CHUNKS = N_PILLARS // PCHUNK
K1_GB = 6
K1_STEPS = PCHUNK // (LANES * K1_GB)

CROWS = 8
CP = CROWS * NY
K2_CHUNKS = NX // CROWS
K2_NCH = B * K2_CHUNKS
K2_GROUPS = NY // LANES

_mesh = plsc.VectorSubcoreMesh(core_axis_name="c", subcore_axis_name="s")
_sc_params = pltpu.CompilerParams(needs_layout_passes=False)


def _wid():
    return lax.axis_index("s") * NC + lax.axis_index("c")


K0_BLK = 1024


def _tr_body(vf_ref, out_ref):
    i = pl.program_id(0)
    rows = i * K0_BLK + lax.broadcasted_iota(jnp.int32, (K0_BLK, 1), 0)
    x = jnp.where(rows < N_PILLARS, vf_ref[...], 0.0)
    out_ref[...] = x.T


def _transpose_table(vf):
    return pl.pallas_call(
        _tr_body,
        grid=(TW // K0_BLK,),
        in_specs=[pl.BlockSpec((K0_BLK, C), lambda i: (i, 0))],
        out_specs=pl.BlockSpec((C, K0_BLK), lambda i: (0, i)),
        out_shape=jax.ShapeDtypeStruct((C, TW), jnp.float32),
    )(vf)


@functools.partial(
    pl.kernel,
    out_type=jax.ShapeDtypeStruct((TOT,), jnp.int32),
    mesh=_mesh,
    compiler_params=_sc_params,
    scratch_types=[
        pltpu.VMEM((POS_PER_W,), jnp.int32),
        pltpu.VMEM((2, PCHUNK), jnp.int32),
        pltpu.VMEM((2, PCHUNK), jnp.int32),
        pltpu.VMEM((2, PCHUNK), jnp.int32),
        pltpu.SemaphoreType.DMA,
        pltpu.SemaphoreType.DMA,
    ],
)
def _build_pid(b_hbm, y_hbm, x_hbm, pid_hbm, slab, bv, yv, xv, si0, si1):
    wid = _wid()
    lo = wid * POS_PER_W
    lane = lax.iota(jnp.int32, LANES)
    sent_vec = SENT + lane
    sin = (si0, si1)

    def start_in(ci, q):
        base = ci * PCHUNK
        pltpu.async_copy(b_hbm.at[pl.ds(base, PCHUNK)], bv.at[q], sin[q])
        pltpu.async_copy(y_hbm.at[pl.ds(base, PCHUNK)], yv.at[q], sin[q])
        pltpu.async_copy(x_hbm.at[pl.ds(base, PCHUNK)], xv.at[q], sin[q])

    def wait_in(q):
        pltpu.make_async_copy(b_hbm.at[pl.ds(0, PCHUNK)], bv.at[q], sin[q]).wait()
        pltpu.make_async_copy(y_hbm.at[pl.ds(0, PCHUNK)], yv.at[q], sin[q]).wait()
        pltpu.make_async_copy(x_hbm.at[pl.ds(0, PCHUNK)], xv.at[q], sin[q]).wait()

    start_in(0, 0)
    start_in(1, 1)

    def ibody(i, carry):
        slab[pl.ds(i * LANES, LANES)] = sent_vec
        return carry

    lax.fori_loop(0, POS_PER_W // LANES, ibody, 0, unroll=8)

    for ci in range(K1_CHUNKS):
        q = ci & 1
        base = ci * PCHUNK
        wait_in(q)

        def gbody(t, carry):
            offs = [(t * K1_GB + j) * LANES for j in range(K1_GB)]
            bbs = [bv[q, pl.ds(o, LANES)] for o in offs]
            yys = [yv[q, pl.ds(o, LANES)] for o in offs]
            xxs = [xv[q, pl.ds(o, LANES)] for o in offs]
            for o, bb, yy, xx in zip(offs, bbs, yys, xxs):
                flat = bb * PLANE + xx * NY + yy
                loc = flat - lo
                mask = (loc >= 0) & (loc < POS_PER_W)
                safe = jnp.where(mask, loc, 0)
                pidv = (base + o) + lane
                plsc.store_scatter(slab, [safe], pidv, mask=mask)
            return carry

        lax.fori_loop(0, K1_STEPS, gbody, 0)
        if ci + 2 < K1_CHUNKS:
            start_in(ci + 2, q)

    pltpu.sync_copy(slab, pid_hbm.at[pl.ds(lo, POS_PER_W)])


@functools.partial(
    pl.kernel,
    out_type=jax.ShapeDtypeStruct((B, C, NX, NY), jnp.float32),
    mesh=_mesh,
    compiler_params=_sc_params,
    scratch_types=[
        pltpu.VMEM((TW,), jnp.float32),
        pltpu.VMEM((TW,), jnp.float32),
        pltpu.VMEM((2, CP), jnp.int32),
        pltpu.VMEM((2, CROWS, NY), jnp.float32),
        pltpu.VMEM((2, CROWS, NY), jnp.float32),
        pltpu.SemaphoreType.DMA,
        pltpu.SemaphoreType.DMA,
        pltpu.SemaphoreType.DMA,
        pltpu.SemaphoreType.DMA,
    ],
)
def _expand(pid_hbm, tab_hbm, out_hbm, t0, t1, pidb, ob0, ob1, si0, si1, so0, so1):
    wid = _wid()
    c0 = wid * 2
    pltpu.sync_copy(tab_hbm.at[c0], t0)
    pltpu.sync_copy(tab_hbm.at[c0 + 1], t1)

    sin = (si0, si1)
    sout = (so0, so1)

    def bk(i):
        b = i // K2_CHUNKS
        return b, i - b * K2_CHUNKS

    def start_in(i, q):
        b, k = bk(i)
        pltpu.async_copy(pid_hbm.at[pl.ds(b * PLANE + k * CP, CP)], pidb.at[q], sin[q])

    def wait_in(q):
        pltpu.make_async_copy(pid_hbm.at[pl.ds(0, CP)], pidb.at[q], sin[q]).wait()

    def start_out(i, q):
        b, k = bk(i)
        pltpu.async_copy(ob0.at[q], out_hbm.at[b, c0, pl.ds(k * CROWS, CROWS)], sout[q])
        pltpu.async_copy(ob1.at[q], out_hbm.at[b, c0 + 1, pl.ds(k * CROWS, CROWS)], sout[q])

    def wait_out(q):
        pltpu.make_async_copy(ob0.at[q], out_hbm.at[0, 0, pl.ds(0, CROWS)], sout[q]).wait()
        pltpu.make_async_copy(ob1.at[q], out_hbm.at[0, 0, pl.ds(0, CROWS)], sout[q]).wait()

    def compute(q):
        def rbody(r, carry):
            rb = r * NY
            for q0, qn in ((0, 8), (8, 8), (16, 8), (24, 7)):
                offs = [(q0 + j) * LANES for j in range(qn)]
                idxs = [pidb[q, pl.ds(rb + o, LANES)] for o in offs]
                v0s = [plsc.load_gather(t0, [ix]) for ix in idxs]
                v1s = [plsc.load_gather(t1, [ix]) for ix in idxs]
                for o, v0, v1 in zip(offs, v0s, v1s):
                    ob0[q, r, pl.ds(o, LANES)] = v0
                    ob1[q, r, pl.ds(o, LANES)] = v1
            return carry

        lax.fori_loop(0, CROWS, rbody, 0, unroll=2)

    start_in(0, 0)
    start_in(1, 1)
    for q in (0, 1):
        wait_in(q)
        compute(q)
        start_out(q, q)
        start_in(q + 2, q)

    def pbody(j, carry):
        for q in (0, 1):
            i = 2 + 2 * j + q
            wait_in(q)
            wait_out(q)
            compute(q)
            start_out(i, q)
            start_in(jnp.minimum(i + 2, K2_NCH - 1), q)
        return carry

    lax.fori_loop(0, (K2_NCH - 2) // 2, pbody, 0)

    for q in (0, 1):
        wait_in(q)
        wait_out(q)


def kernel(voxel_features, batch_idx, y_idx, x_idx):
    vf = voxel_features.astype(jnp.float32)
    bi = batch_idx.astype(jnp.int32)
    yi = y_idx.astype(jnp.int32)
    xi = x_idx.astype(jnp.int32)

    table = _transpose_table(vf)
    pid = _build_pid(bi, yi, xi)
    return jnp.swapaxes(_expand(pid, table), 2, 3)
